# baseline (speedup 1.0000x reference)
import jax
import jax.numpy as jnp
from jax.experimental import pallas as pl
from jax.experimental.pallas import tpu as pltpu

VOCAB_SIZE = 8192
ROWS_PER_STEP = 8


def _body(idx_ref, tgt_ref, *refs):
    row_refs = refs[:ROWS_PER_STEP]
    out_ref, loss_ref = refs[ROWS_PER_STEP], refs[ROWS_PER_STEP + 1]
    i = pl.program_id(0)

    @pl.when(i == 0)
    def _():
        loss_ref[0, 0] = 0.0

    lane = jax.lax.broadcasted_iota(jnp.int32, (1, VOCAB_SIZE), 1)
    nll = 0.0
    for k in range(ROWS_PER_STEP):
        row = row_refs[k][0]
        out_ref[k : k + 1, :] = row
        m = jnp.max(row)
        s = jnp.sum(jnp.exp(row - m))
        t = tgt_ref[i * ROWS_PER_STEP + k]
        tv = jnp.sum(jnp.where(lane == t, row, 0.0))
        nll += m + jnp.log(s) - tv
    loss_ref[0, 0] += nll

    @pl.when(i == pl.num_programs(0) - 1)
    def _():
        loss_ref[0, 0] = loss_ref[0, 0] / (pl.num_programs(0) * ROWS_PER_STEP)


def kernel(idx, targets, token_embedding_table):
    B, T = idx.shape
    n = B * T
    idx_flat = idx.reshape(n).astype(jnp.int32)
    tgt_flat = targets.reshape(n).astype(jnp.int32)
    grid = n // ROWS_PER_STEP

    table3 = token_embedding_table.reshape(VOCAB_SIZE, 1, VOCAB_SIZE)

    def row_spec(k):
        return pl.BlockSpec(
            (1, 1, VOCAB_SIZE),
            lambda i, idx_ref, tgt_ref, k=k: (idx_ref[i * ROWS_PER_STEP + k], 0, 0),
        )

    logits_flat, loss = pl.pallas_call(
        _body,
        grid_spec=pltpu.PrefetchScalarGridSpec(
            num_scalar_prefetch=2,
            grid=(grid,),
            in_specs=[row_spec(k) for k in range(ROWS_PER_STEP)],
            out_specs=[
                pl.BlockSpec(
                    (ROWS_PER_STEP, VOCAB_SIZE),
                    lambda i, idx_ref, tgt_ref: (i, 0),
                ),
                pl.BlockSpec(memory_space=pltpu.SMEM),
            ],
        ),
        out_shape=[
            jax.ShapeDtypeStruct((n, VOCAB_SIZE), jnp.float32),
            jax.ShapeDtypeStruct((1, 1), jnp.float32),
        ],
    )(idx_flat, tgt_flat, *([table3] * ROWS_PER_STEP))

    return logits_flat.reshape(B, T, VOCAB_SIZE), loss[0, 0]


# speedup vs baseline: 1.1695x; 1.1695x over previous
import jax
import jax.numpy as jnp
from jax.experimental import pallas as pl
from jax.experimental.pallas import tpu as pltpu

VOCAB_SIZE = 8192
SUB = 8
LANES = VOCAB_SIZE // SUB
ROWS_PER_STEP = 16


def _body(idx_ref, tgt_ref, *refs):
    row_refs = refs[:ROWS_PER_STEP]
    out_ref, loss_ref = refs[ROWS_PER_STEP], refs[ROWS_PER_STEP + 1]
    i = pl.program_id(0)

    @pl.when(i == 0)
    def _():
        loss_ref[0, 0] = 0.0

    pos = (
        jax.lax.broadcasted_iota(jnp.int32, (SUB, LANES), 0) * LANES
        + jax.lax.broadcasted_iota(jnp.int32, (SUB, LANES), 1)
    )
    nll = 0.0
    for k in range(ROWS_PER_STEP):
        x = row_refs[k][0]
        out_ref[k * SUB : (k + 1) * SUB, :] = x
        m = jnp.max(x)
        s = jnp.sum(jnp.exp(x - m))
        t = tgt_ref[i * ROWS_PER_STEP + k]
        tv = jnp.sum(jnp.where(pos == t, x, 0.0))
        nll += m + jnp.log(s) - tv
    loss_ref[0, 0] += nll

    @pl.when(i == pl.num_programs(0) - 1)
    def _():
        loss_ref[0, 0] = loss_ref[0, 0] / (pl.num_programs(0) * ROWS_PER_STEP)


def kernel(idx, targets, token_embedding_table):
    B, T = idx.shape
    n = B * T
    idx_flat = idx.reshape(n).astype(jnp.int32)
    tgt_flat = targets.reshape(n).astype(jnp.int32)
    grid = n // ROWS_PER_STEP

    table4 = token_embedding_table.reshape(VOCAB_SIZE, SUB, LANES)

    def row_spec(k):
        return pl.BlockSpec(
            (1, SUB, LANES),
            lambda i, idx_ref, tgt_ref, k=k: (idx_ref[i * ROWS_PER_STEP + k], 0, 0),
        )

    logits_flat, loss = pl.pallas_call(
        _body,
        grid_spec=pltpu.PrefetchScalarGridSpec(
            num_scalar_prefetch=2,
            grid=(grid,),
            in_specs=[row_spec(k) for k in range(ROWS_PER_STEP)],
            out_specs=[
                pl.BlockSpec(
                    (ROWS_PER_STEP * SUB, LANES),
                    lambda i, idx_ref, tgt_ref: (i, 0),
                ),
                pl.BlockSpec(memory_space=pltpu.SMEM),
            ],
        ),
        out_shape=[
            jax.ShapeDtypeStruct((n * SUB, LANES), jnp.float32),
            jax.ShapeDtypeStruct((1, 1), jnp.float32),
        ],
    )(idx_flat, tgt_flat, *([table4] * ROWS_PER_STEP))

    return logits_flat.reshape(B, T, VOCAB_SIZE), loss[0, 0]


# speedup vs baseline: 1.9375x; 1.6566x over previous
import jax
import jax.numpy as jnp
from jax.experimental import pallas as pl
from jax.experimental.pallas import tpu as pltpu

VOCAB_SIZE = 8192
SUB = 8
LANES = VOCAB_SIZE // SUB
ROWS_PER_STEP = 16


def _body(idx_ref, tgt_ref, *refs):
    row_refs = refs[:ROWS_PER_STEP]
    out_ref, loss_ref = refs[ROWS_PER_STEP], refs[ROWS_PER_STEP + 1]
    i = pl.program_id(0)

    @pl.when(i == 0)
    def _():
        loss_ref[0, 0] = 0.0

    pos = (
        jax.lax.broadcasted_iota(jnp.int32, (SUB, LANES), 0) * LANES
        + jax.lax.broadcasted_iota(jnp.int32, (SUB, LANES), 1)
    )
    tv_acc = jnp.zeros((SUB, LANES), jnp.float32)
    for k in range(ROWS_PER_STEP):
        x = row_refs[k][0]
        out_ref[k * SUB : (k + 1) * SUB, :] = x
        t = tgt_ref[i * ROWS_PER_STEP + k]
        tv_acc += jnp.where(pos == t, x, 0.0)
    tv_sum = jnp.sum(tv_acc)

    X = out_ref[...]
    mg = jnp.max(X)
    E = jnp.exp(X - mg)
    ones = jnp.ones((LANES, 128), jnp.float32)
    R1 = jax.lax.dot_general(
        E, ones, (((1,), (0,)), ((), ())),
        preferred_element_type=jnp.float32,
        precision=jax.lax.Precision.HIGHEST,
    )
    G = (
        jax.lax.broadcasted_iota(jnp.int32, (ROWS_PER_STEP, ROWS_PER_STEP * SUB), 1)
        // SUB
        == jax.lax.broadcasted_iota(
            jnp.int32, (ROWS_PER_STEP, ROWS_PER_STEP * SUB), 0
        )
    ).astype(jnp.float32)
    R2 = jax.lax.dot_general(
        G, R1, (((1,), (0,)), ((), ())),
        preferred_element_type=jnp.float32,
        precision=jax.lax.Precision.HIGHEST,
    )
    s_col = R2[:, 0:1]
    sum_log_s = jnp.sum(jnp.log(s_col))
    loss_ref[0, 0] += ROWS_PER_STEP * mg + sum_log_s - tv_sum

    @pl.when(i == pl.num_programs(0) - 1)
    def _():
        loss_ref[0, 0] = loss_ref[0, 0] / (pl.num_programs(0) * ROWS_PER_STEP)


def kernel(idx, targets, token_embedding_table):
    B, T = idx.shape
    n = B * T
    idx_flat = idx.reshape(n).astype(jnp.int32)
    tgt_flat = targets.reshape(n).astype(jnp.int32)
    grid = n // ROWS_PER_STEP

    table4 = token_embedding_table.reshape(VOCAB_SIZE, SUB, LANES)

    def row_spec(k):
        return pl.BlockSpec(
            (1, SUB, LANES),
            lambda i, idx_ref, tgt_ref, k=k: (idx_ref[i * ROWS_PER_STEP + k], 0, 0),
        )

    logits_flat, loss = pl.pallas_call(
        _body,
        grid_spec=pltpu.PrefetchScalarGridSpec(
            num_scalar_prefetch=2,
            grid=(grid,),
            in_specs=[row_spec(k) for k in range(ROWS_PER_STEP)],
            out_specs=[
                pl.BlockSpec(
                    (ROWS_PER_STEP * SUB, LANES),
                    lambda i, idx_ref, tgt_ref: (i, 0),
                ),
                pl.BlockSpec(memory_space=pltpu.SMEM),
            ],
        ),
        out_shape=[
            jax.ShapeDtypeStruct((n * SUB, LANES), jnp.float32),
            jax.ShapeDtypeStruct((1, 1), jnp.float32),
        ],
    )(idx_flat, tgt_flat, *([table4] * ROWS_PER_STEP))

    return logits_flat.reshape(B, T, VOCAB_SIZE), loss[0, 0]


# speedup vs baseline: 2.5529x; 1.3176x over previous
import jax
import jax.numpy as jnp
from jax.experimental import pallas as pl
from jax.experimental.pallas import tpu as pltpu

VOCAB_SIZE = 8192
SUB = 8
LANES = VOCAB_SIZE // SUB
ROWS_PER_STEP = 16


def _body(idx_ref, tgt_ref, *refs):
    row_refs = refs[:ROWS_PER_STEP]
    out_ref, loss_ref = refs[ROWS_PER_STEP], refs[ROWS_PER_STEP + 1]
    i = pl.program_id(0)

    @pl.when(i == 0)
    def _():
        loss_ref[0, 0] = 0.0

    pos = (
        jax.lax.broadcasted_iota(jnp.int32, (SUB, LANES), 0) * LANES
        + jax.lax.broadcasted_iota(jnp.int32, (SUB, LANES), 1)
    )
    tv_acc = jnp.zeros((SUB, LANES), jnp.float32)
    for k in range(ROWS_PER_STEP):
        x = row_refs[k][0]
        out_ref[k * SUB : (k + 1) * SUB, :] = x
        t = tgt_ref[i * ROWS_PER_STEP + k]
        tv_acc += jnp.where(pos == t, x, 0.0)
    tv_sum = jnp.sum(tv_acc)

    X = out_ref[...]
    mg = jnp.max(X)
    E = jnp.exp(X - mg)
    s_vec = jnp.sum(E.reshape(ROWS_PER_STEP, SUB, LANES), axis=(1, 2))
    sum_log_s = jnp.sum(jnp.log(s_vec))
    loss_ref[0, 0] += ROWS_PER_STEP * mg + sum_log_s - tv_sum

    @pl.when(i == pl.num_programs(0) - 1)
    def _():
        loss_ref[0, 0] = loss_ref[0, 0] / (pl.num_programs(0) * ROWS_PER_STEP)


def kernel(idx, targets, token_embedding_table):
    B, T = idx.shape
    n = B * T
    idx_flat = idx.reshape(n).astype(jnp.int32)
    tgt_flat = targets.reshape(n).astype(jnp.int32)
    grid = n // ROWS_PER_STEP

    table4 = token_embedding_table.reshape(VOCAB_SIZE, SUB, LANES)

    def row_spec(k):
        return pl.BlockSpec(
            (1, SUB, LANES),
            lambda i, idx_ref, tgt_ref, k=k: (idx_ref[i * ROWS_PER_STEP + k], 0, 0),
        )

    logits_flat, loss = pl.pallas_call(
        _body,
        grid_spec=pltpu.PrefetchScalarGridSpec(
            num_scalar_prefetch=2,
            grid=(grid,),
            in_specs=[row_spec(k) for k in range(ROWS_PER_STEP)],
            out_specs=[
                pl.BlockSpec(
                    (ROWS_PER_STEP * SUB, LANES),
                    lambda i, idx_ref, tgt_ref: (i, 0),
                ),
                pl.BlockSpec(memory_space=pltpu.SMEM),
            ],
        ),
        out_shape=[
            jax.ShapeDtypeStruct((n * SUB, LANES), jnp.float32),
            jax.ShapeDtypeStruct((1, 1), jnp.float32),
        ],
    )(idx_flat, tgt_flat, *([table4] * ROWS_PER_STEP))

    return logits_flat.reshape(B, T, VOCAB_SIZE), loss[0, 0]
